# Initial kernel scaffold; baseline (speedup 1.0000x reference)
import jax
import jax.numpy as jnp
from jax.experimental import pallas as pl


def kernel(input, target):
    raise NotImplementedError("write your pallas kernel here")


# speedup vs baseline: 2.5242x; 2.5242x over previous
import math

import jax
import jax.numpy as jnp
from jax.experimental import pallas as pl
from jax.experimental.pallas import tpu as pltpu

_N = 4096
_C = 32000
_BN = 32

_MEAN = 0.5
_VAR = 0.1 * math.e
_INV_DENOM = 1.0 / (2.0 * _VAR * _VAR)


def _gwloss_block(x_ref, t_ref, out_ref):
    x = x_ref[...]
    t = t_ref[0]

    m = jnp.max(x, axis=1, keepdims=True)
    s = jnp.sum(jnp.exp(x - m), axis=1, keepdims=True)
    lse = m + jnp.log(s)

    ids = jax.lax.broadcasted_iota(jnp.int32, x.shape, 1)
    safe_t = jnp.maximum(t, 0)
    x_t = jnp.sum(jnp.where(ids == safe_t, x, 0.0), axis=1,
                  keepdims=True)

    logpt = x_t - lse
    pt = jnp.exp(logpt)
    g = jnp.exp(-((pt - _MEAN) ** 2) * _INV_DENOM)
    contrib = jnp.where(t != -1, (g - 0.1 * pt) * logpt, 0.0)

    out_ref[...] = jnp.sum(contrib, axis=0, keepdims=True)[None]


def kernel(input, target):
    n, c = input.shape
    assert n == _N and c == _C
    r = n // _BN
    t3 = target.astype(jnp.int32).reshape(r, _BN, 1)

    partials = pl.pallas_call(
        _gwloss_block,
        grid=(r,),
        in_specs=[
            pl.BlockSpec((_BN, c), lambda i: (i, 0)),
            pl.BlockSpec((1, _BN, 1), lambda i: (i, 0, 0)),
        ],
        out_specs=pl.BlockSpec((1, 1, 1), lambda i: (i, 0, 0)),
        out_shape=jax.ShapeDtypeStruct((r, 1, 1), jnp.float32),
        compiler_params=pltpu.CompilerParams(
            dimension_semantics=("parallel",),
            vmem_limit_bytes=48 * 1024 * 1024,
        ),
    )(input, t3)

    num_valid = jnp.sum(target != -1).astype(jnp.float32)
    return -jnp.sum(partials) / num_valid


# speedup vs baseline: 3.2084x; 1.2710x over previous
import math

import jax
import jax.numpy as jnp
from jax.experimental import pallas as pl
from jax.experimental.pallas import tpu as pltpu

_N = 4096
_C = 32000
_BN = 64

_MEAN = 0.5
_VAR = 0.1 * math.e
_INV_DENOM = 1.0 / (2.0 * _VAR * _VAR)


def _gwloss_block(x_ref, t_ref, out_ref):
    x = x_ref[...]
    t = t_ref[0]

    m = jnp.max(x, axis=1, keepdims=True)
    s = jnp.sum(jnp.exp(x - m), axis=1, keepdims=True)
    lse = m + jnp.log(s)

    ids = jax.lax.broadcasted_iota(jnp.int32, x.shape, 1)
    safe_t = jnp.maximum(t, 0)
    x_t = jnp.sum(jnp.where(ids == safe_t, x, 0.0), axis=1,
                  keepdims=True)

    logpt = x_t - lse
    pt = jnp.exp(logpt)
    g = jnp.exp(-((pt - _MEAN) ** 2) * _INV_DENOM)
    contrib = jnp.where(t != -1, (g - 0.1 * pt) * logpt, 0.0)

    out_ref[...] = jnp.sum(contrib, axis=0, keepdims=True)[None]


def kernel(input, target):
    n, c = input.shape
    assert n == _N and c == _C
    r = n // _BN
    t3 = target.astype(jnp.int32).reshape(r, _BN, 1)

    partials = pl.pallas_call(
        _gwloss_block,
        grid=(r,),
        in_specs=[
            pl.BlockSpec((_BN, c), lambda i: (i, 0)),
            pl.BlockSpec((1, _BN, 1), lambda i: (i, 0, 0)),
        ],
        out_specs=pl.BlockSpec((1, 1, 1), lambda i: (i, 0, 0)),
        out_shape=jax.ShapeDtypeStruct((r, 1, 1), jnp.float32),
        compiler_params=pltpu.CompilerParams(
            dimension_semantics=("parallel",),
            vmem_limit_bytes=48 * 1024 * 1024,
        ),
    )(input, t3)

    num_valid = jnp.sum(target != -1).astype(jnp.float32)
    return -jnp.sum(partials) / num_valid


# speedup vs baseline: 3.5037x; 1.0920x over previous
import math

import jax
import jax.numpy as jnp
from jax.experimental import pallas as pl
from jax.experimental.pallas import tpu as pltpu

_N = 4096
_C = 32000
_BN = 128

_MEAN = 0.5
_VAR = 0.1 * math.e
_INV_DENOM = 1.0 / (2.0 * _VAR * _VAR)


def _gwloss_block(x_ref, t_ref, out_ref):
    x = x_ref[...]
    t = t_ref[0]

    m = jnp.max(x, axis=1, keepdims=True)
    s = jnp.sum(jnp.exp(x - m), axis=1, keepdims=True)
    lse = m + jnp.log(s)

    ids = jax.lax.broadcasted_iota(jnp.int32, x.shape, 1)
    safe_t = jnp.maximum(t, 0)
    x_t = jnp.sum(jnp.where(ids == safe_t, x, 0.0), axis=1,
                  keepdims=True)

    logpt = x_t - lse
    pt = jnp.exp(logpt)
    g = jnp.exp(-((pt - _MEAN) ** 2) * _INV_DENOM)
    contrib = jnp.where(t != -1, (g - 0.1 * pt) * logpt, 0.0)

    out_ref[...] = jnp.sum(contrib, axis=0, keepdims=True)[None]


def kernel(input, target):
    n, c = input.shape
    assert n == _N and c == _C
    r = n // _BN
    t3 = target.astype(jnp.int32).reshape(r, _BN, 1)

    partials = pl.pallas_call(
        _gwloss_block,
        grid=(r,),
        in_specs=[
            pl.BlockSpec((_BN, c), lambda i: (i, 0)),
            pl.BlockSpec((1, _BN, 1), lambda i: (i, 0, 0)),
        ],
        out_specs=pl.BlockSpec((1, 1, 1), lambda i: (i, 0, 0)),
        out_shape=jax.ShapeDtypeStruct((r, 1, 1), jnp.float32),
        compiler_params=pltpu.CompilerParams(
            dimension_semantics=("parallel",),
            vmem_limit_bytes=48 * 1024 * 1024,
        ),
    )(input, t3)

    num_valid = jnp.sum(target != -1).astype(jnp.float32)
    return -jnp.sum(partials) / num_valid
